# baseline (speedup 1.0000x reference)
import jax
import jax.numpy as jnp
from jax.experimental import pallas as pl
from jax.experimental.pallas import tpu as pltpu

_B = 4
_R = 4096
_D = 2048
_SUB = 256
_NSUB = _R // _SUB
_LAG = 4


def _in_copy(x_ref, buf, sem_in, g, s):
    return pltpu.make_async_copy(
        x_ref.at[g, pl.ds(s * _SUB, _SUB)],
        buf.at[pl.ds(s * _SUB, _SUB)],
        sem_in.at[s],
    )


def _out_copy(o_ref, buf, sem_out, g, s):
    return pltpu.make_async_copy(
        buf.at[pl.ds(s * _SUB, _SUB)],
        o_ref.at[g, pl.ds(s * _SUB, _SUB)],
        sem_out.at[s],
    )


def _body(k_ref, x_ref, o_ref, buf, norms_ref, thr_ref, sem_in, sem_out):
    g = pl.program_id(0)

    def mask_and_write(s):
        rows = pl.ds(s * _SUB, _SUB)
        mask = (norms_ref[rows, :] >= thr_ref[0]).astype(jnp.float32)
        buf[rows, :] = buf[rows, :] * mask
        _out_copy(o_ref, buf, sem_out, g - 1, s).start(priority=s % 2)

    def start_read(s):
        @pl.when(g > 0)
        def _release():
            _out_copy(o_ref, buf, sem_out, g - 1, s).wait()

        _in_copy(x_ref, buf, sem_in, g, s).start(priority=s % 2)

    for s in range(_NSUB + _LAG):
        if s < _NSUB:
            sw = s

            @pl.when(g > 0)
            def _mw():
                mask_and_write(sw)

        if s >= _LAG:
            sr = s - _LAG

            @pl.when(g < _B)
            def _rd():
                start_read(sr)

    @pl.when(g < _B)
    def _norms_and_threshold():
        for s in range(_NSUB):
            _in_copy(x_ref, buf, sem_in, g, s).wait()
            x = buf[pl.ds(s * _SUB, _SUB), :]
            ssq = jnp.sum(x * x, axis=-1, keepdims=True)
            norms_ref[pl.ds(s * _SUB, _SUB), :] = jnp.sqrt(ssq)

        k = k_ref[g]
        bits = jax.lax.bitcast_convert_type(
            norms_ref[...].reshape(_R // 128, 128), jnp.int32
        )

        def step(_, carry):
            lo, hi = carry
            mid = lo + (hi - lo + 1) // 2
            cnt = jnp.sum((bits >= mid).astype(jnp.int32))
            big = cnt >= k
            return jnp.where(big, mid, lo), jnp.where(big, hi, mid - 1)

        lo, _ = jax.lax.fori_loop(
            0, 31, step, (jnp.int32(0), jnp.int32(0x7F800000))
        )
        thr_ref[0] = jax.lax.bitcast_convert_type(lo, jnp.float32)

    @pl.when(g == _B)
    def _drain():
        for s in range(_NSUB):
            _out_copy(o_ref, buf, sem_out, g - 1, s).wait()


@jax.jit
def kernel(weight_params, logits, W_score, b_score):
    keep_fraction_logit = logits @ W_score + b_score
    keep_fraction = jax.nn.sigmoid(keep_fraction_logit)
    rows_to_keep = jnp.maximum((keep_fraction * _R).astype(jnp.int32), 1)
    rows_to_keep = jnp.squeeze(rows_to_keep, axis=-1)

    return pl.pallas_call(
        _body,
        grid=(_B + 1,),
        in_specs=[
            pl.BlockSpec(memory_space=pltpu.SMEM),
            pl.BlockSpec(memory_space=pl.ANY),
        ],
        out_specs=pl.BlockSpec(memory_space=pl.ANY),
        out_shape=jax.ShapeDtypeStruct((_B, _R, _D), jnp.float32),
        scratch_shapes=[
            pltpu.VMEM((_R, _D), jnp.float32),
            pltpu.VMEM((_R, 1), jnp.float32),
            pltpu.SMEM((1,), jnp.float32),
            pltpu.SemaphoreType.DMA((_NSUB,)),
            pltpu.SemaphoreType.DMA((_NSUB,)),
        ],
    )(rows_to_keep, weight_params)


# speedup vs baseline: 1.2290x; 1.2290x over previous
import jax
import jax.numpy as jnp
from jax.experimental import pallas as pl
from jax.experimental.pallas import tpu as pltpu

_B = 4
_R = 4096
_D = 2048
_SUB = 256
_NSUB = _R // _SUB
_LAG = 4


def _in_copy(x_ref, buf, sem_in, g, s):
    return pltpu.make_async_copy(
        x_ref.at[g, pl.ds(s * _SUB, _SUB)],
        buf.at[pl.ds(s * _SUB, _SUB)],
        sem_in.at[s],
    )


def _out_copy(o_ref, buf, sem_out, g, s):
    return pltpu.make_async_copy(
        buf.at[pl.ds(s * _SUB, _SUB)],
        o_ref.at[g, pl.ds(s * _SUB, _SUB)],
        sem_out.at[s],
    )


def _body(k_ref, x_ref, o_ref, buf, norms_ref, thr_ref, sem_in, sem_out):
    g = pl.program_id(0)

    def mask_and_write(s):
        rows = pl.ds(s * _SUB, _SUB)
        mask = (norms_ref[rows, :] >= thr_ref[0]).astype(jnp.float32)
        buf[rows, :] = buf[rows, :] * mask
        _out_copy(o_ref, buf, sem_out, g - 1, s).start(priority=s % 2)

    def start_read(s):
        @pl.when(g > 0)
        def _release():
            _out_copy(o_ref, buf, sem_out, g - 1, s).wait()

        _in_copy(x_ref, buf, sem_in, g, s).start(priority=s % 2)

    for s in range(_NSUB + _LAG):
        if s < _NSUB:
            sw = s

            @pl.when(g > 0)
            def _mw():
                mask_and_write(sw)

        if s >= _LAG:
            sr = s - _LAG

            @pl.when(g < _B)
            def _rd():
                start_read(sr)

    @pl.when(g < _B)
    def _norms_and_threshold():
        for s in range(_NSUB):
            _in_copy(x_ref, buf, sem_in, g, s).wait()
            x = buf[pl.ds(s * _SUB, _SUB), :]
            ssq = jnp.sum(x * x, axis=-1, keepdims=True)
            norms_ref[pl.ds(s * _SUB, _SUB), :] = jnp.sqrt(ssq)

        kv = jnp.full((1, 128), k_ref[g], jnp.float32)
        bits = jax.lax.bitcast_convert_type(
            norms_ref[...].reshape(_R // 128, 128), jnp.int32
        )
        ones = jnp.ones((128, 128), jnp.float32)

        def step(_, carry):
            lo, hi = carry
            mid = lo + (hi - lo + 1) // 2
            ind = jnp.where(bits >= mid, 1.0, 0.0)
            part = (
                ind[0:8, :] + ind[8:16, :] + ind[16:24, :] + ind[24:32, :]
            )
            rowsum = jax.lax.dot_general(
                part, ones, (((1,), (0,)), ((), ())),
                preferred_element_type=jnp.float32,
            )
            cnt = rowsum + pltpu.roll(rowsum, 4, axis=0)
            cnt = cnt + pltpu.roll(cnt, 2, axis=0)
            cnt = cnt + pltpu.roll(cnt, 1, axis=0)
            big = cnt[0:1, :] >= kv
            return jnp.where(big, mid, lo), jnp.where(big, hi, mid - 1)

        lo, _ = jax.lax.fori_loop(
            0, 31, step,
            (jnp.zeros((1, 128), jnp.int32),
             jnp.full((1, 128), 0x7F800000, jnp.int32)),
        )
        thr_ref[0] = jax.lax.bitcast_convert_type(lo, jnp.float32)[0, 0]

    @pl.when(g == _B)
    def _drain():
        for s in range(_NSUB):
            _out_copy(o_ref, buf, sem_out, g - 1, s).wait()


@jax.jit
def kernel(weight_params, logits, W_score, b_score):
    keep_fraction_logit = logits @ W_score + b_score
    keep_fraction = jax.nn.sigmoid(keep_fraction_logit)
    rows_to_keep = jnp.maximum((keep_fraction * _R).astype(jnp.int32), 1)
    rows_to_keep = jnp.squeeze(rows_to_keep, axis=-1)

    return pl.pallas_call(
        _body,
        grid=(_B + 1,),
        in_specs=[
            pl.BlockSpec(memory_space=pltpu.SMEM),
            pl.BlockSpec(memory_space=pl.ANY),
        ],
        out_specs=pl.BlockSpec(memory_space=pl.ANY),
        out_shape=jax.ShapeDtypeStruct((_B, _R, _D), jnp.float32),
        scratch_shapes=[
            pltpu.VMEM((_R, _D), jnp.float32),
            pltpu.VMEM((_R, 1), jnp.float32),
            pltpu.SMEM((1,), jnp.float32),
            pltpu.SemaphoreType.DMA((_NSUB,)),
            pltpu.SemaphoreType.DMA((_NSUB,)),
        ],
    )(rows_to_keep, weight_params)
